# baseline (speedup 1.0000x reference)
import functools

import jax
import jax.numpy as jnp
from jax import lax
from jax.experimental import pallas as pl
from jax.experimental.pallas import tpu as pltpu
from jax.experimental.pallas import tpu_sc as plsc

_NUM_CORES = 2
_NUM_SUBCORES = 16
_NUM_WORKERS = _NUM_CORES * _NUM_SUBCORES

_TILE_V = 512


def _sc_gather(table, idx):
    v, d = table.shape
    b = idx.shape[0]
    b_per_w = b // _NUM_WORKERS

    mesh = plsc.VectorSubcoreMesh(core_axis_name="c", subcore_axis_name="s")

    @functools.partial(
        pl.kernel,
        out_type=jax.ShapeDtypeStruct((b, d), jnp.float32),
        mesh=mesh,
        scratch_types=[
            pltpu.VMEM((b_per_w,), jnp.int32),
            pltpu.VMEM((b_per_w, d), jnp.float32),
            pltpu.SemaphoreType.DMA,
        ],
        compiler_params=pltpu.CompilerParams(use_tc_tiling_on_sc=False),
    )
    def gather_kernel(table_hbm, idx_hbm, out_hbm, idx_v, rows_v, sem):
        wid = lax.axis_index("s") * _NUM_CORES + lax.axis_index("c")
        base = wid * b_per_w
        pltpu.sync_copy(idx_hbm.at[pl.ds(base, b_per_w)], idx_v)
        pltpu.async_copy(table_hbm.at[idx_v], rows_v, sem).wait()
        pltpu.sync_copy(rows_v, out_hbm.at[pl.ds(base, b_per_w)])

    return gather_kernel(table, idx)


def _tc_project(h, w, bias):
    b_rows, d = h.shape
    v = w.shape[0]
    grid = pl.cdiv(v, _TILE_V)

    def body(h_ref, w_ref, b_ref, o_ref):
        acc = lax.dot_general(
            h_ref[...], w_ref[...],
            (((1,), (1,)), ((), ())),
            preferred_element_type=jnp.float32,
        )
        o_ref[...] = acc + b_ref[...]

    return pl.pallas_call(
        body,
        grid=(grid,),
        in_specs=[
            pl.BlockSpec((b_rows, d), lambda i: (0, 0)),
            pl.BlockSpec((_TILE_V, d), lambda i: (i, 0)),
            pl.BlockSpec((1, _TILE_V), lambda i: (0, i)),
        ],
        out_specs=pl.BlockSpec((b_rows, _TILE_V), lambda i: (0, i)),
        out_shape=jax.ShapeDtypeStruct((b_rows, v), jnp.float32),
        compiler_params=pltpu.CompilerParams(
            dimension_semantics=("arbitrary",),
        ),
    )(h, w, bias.reshape(1, v))


def kernel(x, embed_table, W, b):
    h = _sc_gather(embed_table, x.astype(jnp.int32))
    return _tc_project(h, W, b)


# speedup vs baseline: 2.2633x; 2.2633x over previous
import functools

import jax
import jax.numpy as jnp
from jax import lax
from jax.experimental import pallas as pl
from jax.experimental.pallas import tpu as pltpu
from jax.experimental.pallas import tpu_sc as plsc

_NUM_CORES = 2
_NUM_SUBCORES = 16
_NUM_WORKERS = _NUM_CORES * _NUM_SUBCORES

_TILE_V = 512


def _sc_gather(table, idx):
    v, d = table.shape
    b = idx.shape[0]
    b_per_w = b // _NUM_WORKERS

    mesh = plsc.VectorSubcoreMesh(core_axis_name="c", subcore_axis_name="s")

    @functools.partial(
        pl.kernel,
        out_type=jax.ShapeDtypeStruct((b, d), jnp.float32),
        mesh=mesh,
        scratch_types=[
            pltpu.VMEM((b_per_w,), jnp.int32),
            pltpu.VMEM((b_per_w, d), jnp.float32),
            pltpu.SemaphoreType.DMA,
        ],
        compiler_params=pltpu.CompilerParams(use_tc_tiling_on_sc=False),
    )
    def gather_kernel(table_hbm, idx_hbm, out_hbm, idx_v, rows_v, sem):
        wid = lax.axis_index("s") * _NUM_CORES + lax.axis_index("c")
        base = wid * b_per_w
        pltpu.sync_copy(idx_hbm.at[pl.ds(base, b_per_w)], idx_v)
        pltpu.async_copy(table_hbm.at[idx_v], rows_v, sem).wait()
        pltpu.sync_copy(rows_v, out_hbm.at[pl.ds(base, b_per_w)])

    return gather_kernel(table, idx)


def _tc_project_t(h, wt, bias):
    b_rows = h.shape[0]
    v = wt.shape[1]
    grid = pl.cdiv(v, _TILE_V)

    def body(h_ref, wt_ref, b_ref, o_ref):
        acc = lax.dot_general(
            h_ref[...], wt_ref[...],
            (((1,), (0,)), ((), ())),
            preferred_element_type=jnp.float32,
        )
        o_ref[...] = jnp.transpose(acc + b_ref[...])

    return pl.pallas_call(
        body,
        grid=(grid,),
        in_specs=[
            pl.BlockSpec((b_rows, h.shape[1]), lambda i: (0, 0)),
            pl.BlockSpec((wt.shape[0], _TILE_V), lambda i: (0, i)),
            pl.BlockSpec((1, _TILE_V), lambda i: (0, i)),
        ],
        out_specs=pl.BlockSpec((_TILE_V, b_rows), lambda i: (i, 0)),
        out_shape=jax.ShapeDtypeStruct((v, b_rows), jnp.float32),
        compiler_params=pltpu.CompilerParams(
            dimension_semantics=("arbitrary",),
        ),
    )(h, wt, bias.reshape(1, v))


def kernel(x, embed_table, W, b):
    h = _sc_gather(embed_table, x.astype(jnp.int32))
    out_t = _tc_project_t(h, W.T, b)
    return out_t.T


# speedup vs baseline: 2.8413x; 1.2554x over previous
import functools

import jax
import jax.numpy as jnp
from jax import lax
from jax.experimental import pallas as pl
from jax.experimental.pallas import tpu as pltpu
from jax.experimental.pallas import tpu_sc as plsc

_NUM_CORES = 2
_NUM_SUBCORES = 16
_NUM_WORKERS = _NUM_CORES * _NUM_SUBCORES

_TILE_V = 1024


def _sc_gather(table, idx):
    v, d = table.shape
    b = idx.shape[0]
    b_per_w = b // _NUM_WORKERS

    mesh = plsc.VectorSubcoreMesh(core_axis_name="c", subcore_axis_name="s")

    @functools.partial(
        pl.kernel,
        out_type=jax.ShapeDtypeStruct((b, d), jnp.float32),
        mesh=mesh,
        scratch_types=[
            pltpu.VMEM((b_per_w,), jnp.int32),
            pltpu.VMEM((b_per_w, d), jnp.float32),
            pltpu.SemaphoreType.DMA,
        ],
        compiler_params=pltpu.CompilerParams(use_tc_tiling_on_sc=False),
    )
    def gather_kernel(table_hbm, idx_hbm, out_hbm, idx_v, rows_v, sem):
        wid = lax.axis_index("s") * _NUM_CORES + lax.axis_index("c")
        base = wid * b_per_w
        pltpu.sync_copy(idx_hbm.at[pl.ds(base, b_per_w)], idx_v)
        pltpu.async_copy(table_hbm.at[idx_v], rows_v, sem).wait()
        pltpu.sync_copy(rows_v, out_hbm.at[pl.ds(base, b_per_w)])

    return gather_kernel(table, idx)


def _tc_project_t(h, wt, bias):
    b_rows = h.shape[0]
    v = wt.shape[1]
    grid = pl.cdiv(v, _TILE_V)

    def body(h_ref, wt_ref, b_ref, o_ref):
        acc = lax.dot_general(
            h_ref[...], wt_ref[...],
            (((1,), (0,)), ((), ())),
            preferred_element_type=jnp.float32,
        )
        o_ref[...] = jnp.transpose(acc + b_ref[...])

    return pl.pallas_call(
        body,
        grid=(grid,),
        in_specs=[
            pl.BlockSpec((b_rows, h.shape[1]), lambda i: (0, 0)),
            pl.BlockSpec((wt.shape[0], _TILE_V), lambda i: (0, i)),
            pl.BlockSpec((1, _TILE_V), lambda i: (0, i)),
        ],
        out_specs=pl.BlockSpec((_TILE_V, b_rows), lambda i: (i, 0)),
        out_shape=jax.ShapeDtypeStruct((v, b_rows), jnp.float32),
        compiler_params=pltpu.CompilerParams(
            dimension_semantics=("arbitrary",),
        ),
    )(h, wt, bias.reshape(1, v))


def kernel(x, embed_table, W, b):
    h = _sc_gather(embed_table, x.astype(jnp.int32))
    out_t = _tc_project_t(h, W.T, b)
    return out_t.T


# speedup vs baseline: 3.1418x; 1.1058x over previous
import functools

import jax
import jax.numpy as jnp
from jax import lax
from jax.experimental import pallas as pl
from jax.experimental.pallas import tpu as pltpu
from jax.experimental.pallas import tpu_sc as plsc

_NUM_CORES = 2
_NUM_SUBCORES = 16
_NUM_WORKERS = _NUM_CORES * _NUM_SUBCORES

_TILE_V = 2048


def _sc_gather(table, idx):
    v, d = table.shape
    b = idx.shape[0]
    b_per_w = b // _NUM_WORKERS

    mesh = plsc.VectorSubcoreMesh(core_axis_name="c", subcore_axis_name="s")

    @functools.partial(
        pl.kernel,
        out_type=jax.ShapeDtypeStruct((b, d), jnp.float32),
        mesh=mesh,
        scratch_types=[
            pltpu.VMEM((b_per_w,), jnp.int32),
            pltpu.VMEM((b_per_w, d), jnp.float32),
            pltpu.SemaphoreType.DMA,
        ],
        compiler_params=pltpu.CompilerParams(use_tc_tiling_on_sc=False),
    )
    def gather_kernel(table_hbm, idx_hbm, out_hbm, idx_v, rows_v, sem):
        wid = lax.axis_index("s") * _NUM_CORES + lax.axis_index("c")
        base = wid * b_per_w
        pltpu.sync_copy(idx_hbm.at[pl.ds(base, b_per_w)], idx_v)
        pltpu.async_copy(table_hbm.at[idx_v], rows_v, sem).wait()
        pltpu.sync_copy(rows_v, out_hbm.at[pl.ds(base, b_per_w)])

    return gather_kernel(table, idx)


def _tc_project_t(h, wt, bias):
    b_rows = h.shape[0]
    v = wt.shape[1]
    grid = pl.cdiv(v, _TILE_V)

    def body(h_ref, wt_ref, b_ref, o_ref):
        acc = lax.dot_general(
            h_ref[...], wt_ref[...],
            (((1,), (0,)), ((), ())),
            preferred_element_type=jnp.float32,
        )
        o_ref[...] = jnp.transpose(acc + b_ref[...])

    return pl.pallas_call(
        body,
        grid=(grid,),
        in_specs=[
            pl.BlockSpec((b_rows, h.shape[1]), lambda i: (0, 0)),
            pl.BlockSpec((wt.shape[0], _TILE_V), lambda i: (0, i)),
            pl.BlockSpec((1, _TILE_V), lambda i: (0, i)),
        ],
        out_specs=pl.BlockSpec((_TILE_V, b_rows), lambda i: (i, 0)),
        out_shape=jax.ShapeDtypeStruct((v, b_rows), jnp.float32),
        compiler_params=pltpu.CompilerParams(
            dimension_semantics=("arbitrary",),
        ),
    )(h, wt, bias.reshape(1, v))


def kernel(x, embed_table, W, b):
    h = _sc_gather(embed_table, x.astype(jnp.int32))
    out_t = _tc_project_t(h, W.T, b)
    return out_t.T


# speedup vs baseline: 3.2212x; 1.0253x over previous
import functools

import jax
import jax.numpy as jnp
from jax import lax
from jax.experimental import pallas as pl
from jax.experimental.pallas import tpu as pltpu
from jax.experimental.pallas import tpu_sc as plsc

_NUM_CORES = 2
_NUM_SUBCORES = 16
_NUM_WORKERS = _NUM_CORES * _NUM_SUBCORES

_TILE_V = 4096


def _sc_gather(table, idx):
    v, d = table.shape
    b = idx.shape[0]
    b_per_w = b // _NUM_WORKERS

    mesh = plsc.VectorSubcoreMesh(core_axis_name="c", subcore_axis_name="s")

    @functools.partial(
        pl.kernel,
        out_type=jax.ShapeDtypeStruct((b, d), jnp.float32),
        mesh=mesh,
        scratch_types=[
            pltpu.VMEM((b_per_w,), jnp.int32),
            pltpu.VMEM((b_per_w, d), jnp.float32),
            pltpu.SemaphoreType.DMA,
        ],
        compiler_params=pltpu.CompilerParams(use_tc_tiling_on_sc=False),
    )
    def gather_kernel(table_hbm, idx_hbm, out_hbm, idx_v, rows_v, sem):
        wid = lax.axis_index("s") * _NUM_CORES + lax.axis_index("c")
        base = wid * b_per_w
        pltpu.sync_copy(idx_hbm.at[pl.ds(base, b_per_w)], idx_v)
        pltpu.async_copy(table_hbm.at[idx_v], rows_v, sem).wait()
        pltpu.sync_copy(rows_v, out_hbm.at[pl.ds(base, b_per_w)])

    return gather_kernel(table, idx)


def _tc_project_t(h, wt, bias):
    b_rows = h.shape[0]
    v = wt.shape[1]
    grid = pl.cdiv(v, _TILE_V)

    def body(h_ref, wt_ref, b_ref, o_ref):
        acc = lax.dot_general(
            h_ref[...], wt_ref[...],
            (((1,), (0,)), ((), ())),
            preferred_element_type=jnp.float32,
        )
        o_ref[...] = jnp.transpose(acc + b_ref[...])

    return pl.pallas_call(
        body,
        grid=(grid,),
        in_specs=[
            pl.BlockSpec((b_rows, h.shape[1]), lambda i: (0, 0)),
            pl.BlockSpec((wt.shape[0], _TILE_V), lambda i: (0, i)),
            pl.BlockSpec((1, _TILE_V), lambda i: (0, i)),
        ],
        out_specs=pl.BlockSpec((_TILE_V, b_rows), lambda i: (i, 0)),
        out_shape=jax.ShapeDtypeStruct((v, b_rows), jnp.float32),
        compiler_params=pltpu.CompilerParams(
            dimension_semantics=("arbitrary",),
        ),
    )(h, wt, bias.reshape(1, v))


def kernel(x, embed_table, W, b):
    h = _sc_gather(embed_table, x.astype(jnp.int32))
    out_t = _tc_project_t(h, W.T, b)
    return out_t.T


# speedup vs baseline: 3.7005x; 1.1488x over previous
import functools

import jax
import jax.numpy as jnp
from jax import lax
from jax.experimental import pallas as pl
from jax.experimental.pallas import tpu as pltpu
from jax.experimental.pallas import tpu_sc as plsc

_NUM_CORES = 2
_NUM_SUBCORES = 16
_NUM_WORKERS = _NUM_CORES * _NUM_SUBCORES

_TILE_V = 4096


def _sc_gather_t(table_t, idx):
    d, v = table_t.shape
    b = idx.shape[0]

    mesh = plsc.VectorSubcoreMesh(core_axis_name="c", subcore_axis_name="s")

    @functools.partial(
        pl.kernel,
        out_type=jax.ShapeDtypeStruct((d, b), jnp.float32),
        mesh=mesh,
        scratch_types=[
            pltpu.VMEM((v,), jnp.float32),
            pltpu.VMEM((b,), jnp.int32),
            pltpu.VMEM((b,), jnp.float32),
        ],
        compiler_params=pltpu.CompilerParams(
            use_tc_tiling_on_sc=False, needs_layout_passes=False
        ),
    )
    def gather_kernel(table_hbm, idx_hbm, out_hbm, row_v, idx_v, out_v):
        wid = lax.axis_index("s") * _NUM_CORES + lax.axis_index("c")
        pltpu.sync_copy(table_hbm.at[wid], row_v)
        pltpu.sync_copy(idx_hbm, idx_v)
        for j in range(b // 16):
            vals = plsc.load_gather(row_v, [idx_v[pl.ds(j * 16, 16)]])
            out_v[pl.ds(j * 16, 16)] = vals
        pltpu.sync_copy(out_v, out_hbm.at[wid])

    return gather_kernel(table_t, idx)


def _tc_project_t(h, wt, bias):
    b_rows = h.shape[0]
    v = wt.shape[1]
    grid = pl.cdiv(v, _TILE_V)

    def body(h_ref, wt_ref, b_ref, o_ref):
        acc = lax.dot_general(
            h_ref[...], wt_ref[...],
            (((1,), (0,)), ((), ())),
            preferred_element_type=jnp.float32,
        )
        o_ref[...] = jnp.transpose(acc + b_ref[...])

    return pl.pallas_call(
        body,
        grid=(grid,),
        in_specs=[
            pl.BlockSpec((b_rows, h.shape[1]), lambda i: (0, 0)),
            pl.BlockSpec((wt.shape[0], _TILE_V), lambda i: (0, i)),
            pl.BlockSpec((1, _TILE_V), lambda i: (0, i)),
        ],
        out_specs=pl.BlockSpec((_TILE_V, b_rows), lambda i: (i, 0)),
        out_shape=jax.ShapeDtypeStruct((v, b_rows), jnp.float32),
        compiler_params=pltpu.CompilerParams(
            dimension_semantics=("arbitrary",),
        ),
    )(h, wt, bias.reshape(1, v))


def kernel(x, embed_table, W, b):
    h_t = _sc_gather_t(embed_table.T, x.astype(jnp.int32))
    out_t = _tc_project_t(h_t.T, W.T, b)
    return out_t.T


# speedup vs baseline: 4.0319x; 1.0896x over previous
import functools

import jax
import jax.numpy as jnp
from jax import lax
from jax.experimental import pallas as pl
from jax.experimental.pallas import tpu as pltpu
from jax.experimental.pallas import tpu_sc as plsc

_NUM_CORES = 2
_NUM_SUBCORES = 16
_NUM_WORKERS = _NUM_CORES * _NUM_SUBCORES

_TILE_V = 4096


def _sc_gather_t(table_t, idx):
    d, v = table_t.shape
    b = idx.shape[0]

    mesh = plsc.VectorSubcoreMesh(core_axis_name="c", subcore_axis_name="s")

    @functools.partial(
        pl.kernel,
        out_type=jax.ShapeDtypeStruct((d, b), jnp.float32),
        mesh=mesh,
        scratch_types=[
            pltpu.VMEM((v,), jnp.float32),
            pltpu.VMEM((b,), jnp.int32),
            pltpu.VMEM((b,), jnp.float32),
        ],
        compiler_params=pltpu.CompilerParams(
            use_tc_tiling_on_sc=True, needs_layout_passes=False
        ),
    )
    def gather_kernel(table_hbm, idx_hbm, out_hbm, row_v, idx_v, out_v):
        wid = lax.axis_index("s") * _NUM_CORES + lax.axis_index("c")
        pltpu.sync_copy(table_hbm.at[wid], row_v)
        pltpu.sync_copy(idx_hbm, idx_v)
        for j in range(b // 16):
            vals = plsc.load_gather(row_v, [idx_v[pl.ds(j * 16, 16)]])
            out_v[pl.ds(j * 16, 16)] = vals
        pltpu.sync_copy(out_v, out_hbm.at[wid])

    return gather_kernel(table_t, idx)


def _tc_project_t(h, wt, bias):
    b_rows = h.shape[0]
    v = wt.shape[1]
    grid = pl.cdiv(v, _TILE_V)

    def body(h_ref, wt_ref, b_ref, o_ref):
        acc = lax.dot_general(
            h_ref[...], wt_ref[...],
            (((1,), (0,)), ((), ())),
            preferred_element_type=jnp.float32,
        )
        o_ref[...] = jnp.transpose(acc + b_ref[...])

    return pl.pallas_call(
        body,
        grid=(grid,),
        in_specs=[
            pl.BlockSpec((b_rows, h.shape[1]), lambda i: (0, 0)),
            pl.BlockSpec((wt.shape[0], _TILE_V), lambda i: (0, i)),
            pl.BlockSpec((1, _TILE_V), lambda i: (0, i)),
        ],
        out_specs=pl.BlockSpec((_TILE_V, b_rows), lambda i: (i, 0)),
        out_shape=jax.ShapeDtypeStruct((v, b_rows), jnp.float32),
        compiler_params=pltpu.CompilerParams(
            dimension_semantics=("arbitrary",),
        ),
    )(h, wt, bias.reshape(1, v))


def kernel(x, embed_table, W, b):
    h_t = _sc_gather_t(embed_table.T, x.astype(jnp.int32))
    out_t = _tc_project_t(h_t.T, W.T, b)
    return out_t.T


# speedup vs baseline: 4.0606x; 1.0071x over previous
import functools

import jax
import jax.numpy as jnp
from jax import lax
from jax.experimental import pallas as pl
from jax.experimental.pallas import tpu as pltpu
from jax.experimental.pallas import tpu_sc as plsc

_NUM_CORES = 2
_NUM_SUBCORES = 16
_NUM_WORKERS = _NUM_CORES * _NUM_SUBCORES

_TILE_V = 5120


def _sc_gather_t(table_t, idx):
    d, v = table_t.shape
    b = idx.shape[0]

    mesh = plsc.VectorSubcoreMesh(core_axis_name="c", subcore_axis_name="s")

    @functools.partial(
        pl.kernel,
        out_type=jax.ShapeDtypeStruct((d, b), jnp.float32),
        mesh=mesh,
        scratch_types=[
            pltpu.VMEM((v,), jnp.float32),
            pltpu.VMEM((b,), jnp.int32),
            pltpu.VMEM((b,), jnp.float32),
        ],
        compiler_params=pltpu.CompilerParams(
            use_tc_tiling_on_sc=True, needs_layout_passes=False
        ),
    )
    def gather_kernel(table_hbm, idx_hbm, out_hbm, row_v, idx_v, out_v):
        wid = lax.axis_index("s") * _NUM_CORES + lax.axis_index("c")
        pltpu.sync_copy(table_hbm.at[wid], row_v)
        pltpu.sync_copy(idx_hbm, idx_v)
        for j in range(b // 16):
            vals = plsc.load_gather(row_v, [idx_v[pl.ds(j * 16, 16)]])
            out_v[pl.ds(j * 16, 16)] = vals
        pltpu.sync_copy(out_v, out_hbm.at[wid])

    return gather_kernel(table_t, idx)


def _tc_project_t(h, wt, bias):
    b_rows = h.shape[0]
    v = wt.shape[1]
    grid = pl.cdiv(v, _TILE_V)

    def body(h_ref, wt_ref, b_ref, o_ref):
        acc = lax.dot_general(
            h_ref[...], wt_ref[...],
            (((1,), (0,)), ((), ())),
            preferred_element_type=jnp.float32,
        )
        o_ref[...] = jnp.transpose(acc + b_ref[...])

    return pl.pallas_call(
        body,
        grid=(grid,),
        in_specs=[
            pl.BlockSpec((b_rows, h.shape[1]), lambda i: (0, 0)),
            pl.BlockSpec((wt.shape[0], _TILE_V), lambda i: (0, i)),
            pl.BlockSpec((1, _TILE_V), lambda i: (0, i)),
        ],
        out_specs=pl.BlockSpec((_TILE_V, b_rows), lambda i: (i, 0)),
        out_shape=jax.ShapeDtypeStruct((v, b_rows), jnp.float32),
        compiler_params=pltpu.CompilerParams(
            dimension_semantics=("arbitrary",),
        ),
    )(h, wt, bias.reshape(1, v))


def kernel(x, embed_table, W, b):
    h_t = _sc_gather_t(embed_table.T, x.astype(jnp.int32))
    out_t = _tc_project_t(h_t.T, W.T, b)
    return out_t.T
